# baseline (speedup 1.0000x reference)
import functools

import jax
import jax.numpy as jnp
from jax import lax
from jax.experimental import pallas as pl
from jax.experimental.pallas import tpu as pltpu
from jax.experimental.pallas import tpu_sc as plsc

NUM_CORES = 2
NUM_SUBCORES = 16
NUM_WORKERS = NUM_CORES * NUM_SUBCORES
LANES = 16

BATCH = 4096
HIST = 50
EMB = 32

PADDED = 128
ROWS_PER_WORKER = BATCH // NUM_WORKERS
ROWS_PER_GATHER = 4
IDX_PER_GATHER = ROWS_PER_GATHER * HIST
GATHERS = ROWS_PER_WORKER // ROWS_PER_GATHER

_SCALE = 1.0 / HIST


def _reduce_group(buf, row0, out_v, out_row):
    for half in range(EMB // LANES):
        sl = pl.ds(half * LANES, LANES)
        acc = buf[row0, sl]
        for k in range(1, HIST):
            acc = acc + buf[row0 + k, sl]
        out_v[out_row, sl] = acc * _SCALE


def _emb_mean_body(x_hbm, w_hbm, out_hbm, idx_v, buf_a, buf_b, out_v,
                   sem_a, sem_b):
    wid = lax.axis_index("s") * NUM_CORES + lax.axis_index("c")

    pltpu.sync_copy(x_hbm.at[wid], idx_v)

    def start(j, buf, sem):
        pltpu.make_async_copy(w_hbm.at[idx_v.at[j]], buf, sem).start()

    def wait(j, buf, sem):
        pltpu.make_async_copy(w_hbm.at[idx_v.at[j]], buf, sem).wait()

    def reduce_buf(buf, j):
        for r in range(ROWS_PER_GATHER):
            _reduce_group(buf, r * HIST, out_v, j * ROWS_PER_GATHER + r)

    start(0, buf_a, sem_a)

    @pl.loop(0, GATHERS, step=2)
    def _(j):
        start(j + 1, buf_b, sem_b)
        wait(j, buf_a, sem_a)
        reduce_buf(buf_a, j)

        @pl.when(j + 2 < GATHERS)
        def _():
            start(j + 2, buf_a, sem_a)

        wait(j + 1, buf_b, sem_b)
        reduce_buf(buf_b, j + 1)

    pltpu.sync_copy(out_v, out_hbm.at[pl.ds(wid * ROWS_PER_WORKER,
                                            ROWS_PER_WORKER)])


_TB = 32768
_QT = _TB // 4
_TGRID = (1000000 + _TB - 1) // _TB
_OUT_ROWS = _TGRID * _QT


def _transpose_body(x_ref, o_ref):
    x = x_ref[...]
    stacked = jnp.concatenate(
        [x[:, c * _QT:(c + 1) * _QT] for c in range(4)], axis=0)
    o_ref[...] = stacked.T


def _prepare_table(wt):
    return pl.pallas_call(
        _transpose_body,
        grid=(_TGRID,),
        in_specs=[pl.BlockSpec((EMB, _TB), lambda j: (0, j))],
        out_specs=pl.BlockSpec((_QT, PADDED), lambda j: (j, 0)),
        out_shape=jax.ShapeDtypeStruct((_OUT_ROWS, PADDED), jnp.float32),
    )(wt)


@jax.jit
def _emb_mean(x, w):
    mesh = plsc.VectorSubcoreMesh(core_axis_name="c", subcore_axis_name="s")
    fn = pl.kernel(
        _emb_mean_body,
        out_type=jax.ShapeDtypeStruct((BATCH, EMB), jnp.float32),
        mesh=mesh,
        scratch_types=[
            pltpu.VMEM((GATHERS, IDX_PER_GATHER), jnp.int32),
            pltpu.VMEM((IDX_PER_GATHER, EMB), jnp.float32),
            pltpu.VMEM((IDX_PER_GATHER, EMB), jnp.float32),
            pltpu.VMEM((ROWS_PER_WORKER, EMB), jnp.float32),
            pltpu.SemaphoreType.DMA,
            pltpu.SemaphoreType.DMA,
        ],
        compiler_params=pltpu.CompilerParams(use_tc_tiling_on_sc=False),
    )
    return fn(x, w)


def kernel(X, W):
    xi = X.astype(jnp.int32)
    j, m = xi // _TB, xi % _TB
    c, p = m // _QT, m % _QT
    x = (4 * (j * _QT + p) + c).reshape(NUM_WORKERS, GATHERS, IDX_PER_GATHER)
    w_lin = _prepare_table(W.T).reshape(4 * _OUT_ROWS, EMB)
    return _emb_mean(x, w_lin)


# speedup vs baseline: 1.0599x; 1.0599x over previous
import functools

import jax
import jax.numpy as jnp
from jax import lax
from jax.experimental import pallas as pl
from jax.experimental.pallas import tpu as pltpu
from jax.experimental.pallas import tpu_sc as plsc

NUM_CORES = 2
NUM_SUBCORES = 16
NUM_WORKERS = NUM_CORES * NUM_SUBCORES
LANES = 16

BATCH = 4096
HIST = 50
EMB = 32

PADDED = 128
ROWS_PER_WORKER = BATCH // NUM_WORKERS
ROWS_PER_GATHER = 2
IDX_PER_GATHER = ROWS_PER_GATHER * HIST
GATHERS = ROWS_PER_WORKER // ROWS_PER_GATHER

_SCALE = 1.0 / HIST


def _reduce_group(buf, row0, out_v, out_row):
    for half in range(EMB // LANES):
        sl = pl.ds(half * LANES, LANES)
        acc = buf[row0, sl]
        for k in range(1, HIST):
            acc = acc + buf[row0 + k, sl]
        out_v[out_row, sl] = acc * _SCALE


def _emb_mean_body(x_hbm, w_hbm, out_hbm, idx_v, buf_a, buf_b, out_v,
                   sem_a, sem_b):
    wid = lax.axis_index("s") * NUM_CORES + lax.axis_index("c")

    pltpu.sync_copy(x_hbm.at[wid], idx_v)

    def start(j, buf, sem):
        pltpu.make_async_copy(w_hbm.at[idx_v.at[j]], buf, sem).start()

    def wait(j, buf, sem):
        pltpu.make_async_copy(w_hbm.at[idx_v.at[j]], buf, sem).wait()

    def reduce_buf(buf, j):
        for r in range(ROWS_PER_GATHER):
            _reduce_group(buf, r * HIST, out_v, j * ROWS_PER_GATHER + r)

    start(0, buf_a, sem_a)

    @pl.loop(0, GATHERS, step=2)
    def _(j):
        start(j + 1, buf_b, sem_b)
        wait(j, buf_a, sem_a)
        reduce_buf(buf_a, j)

        @pl.when(j + 2 < GATHERS)
        def _():
            start(j + 2, buf_a, sem_a)

        wait(j + 1, buf_b, sem_b)
        reduce_buf(buf_b, j + 1)

    pltpu.sync_copy(out_v, out_hbm.at[pl.ds(wid * ROWS_PER_WORKER,
                                            ROWS_PER_WORKER)])


_TB = 65536
_QT = _TB // 4
_TGRID = (1000000 + _TB - 1) // _TB
_OUT_ROWS = _TGRID * _QT


def _transpose_body(x_ref, o_ref):
    x = x_ref[...]
    stacked = jnp.concatenate(
        [x[:, c * _QT:(c + 1) * _QT] for c in range(4)], axis=0)
    o_ref[...] = stacked.T


def _prepare_table(wt):
    return pl.pallas_call(
        _transpose_body,
        grid=(_TGRID,),
        in_specs=[pl.BlockSpec((EMB, _TB), lambda j: (0, j))],
        out_specs=pl.BlockSpec((_QT, PADDED), lambda j: (j, 0)),
        out_shape=jax.ShapeDtypeStruct((_OUT_ROWS, PADDED), jnp.float32),
    )(wt)


@jax.jit
def _emb_mean(x, w):
    mesh = plsc.VectorSubcoreMesh(core_axis_name="c", subcore_axis_name="s")
    fn = pl.kernel(
        _emb_mean_body,
        out_type=jax.ShapeDtypeStruct((BATCH, EMB), jnp.float32),
        mesh=mesh,
        scratch_types=[
            pltpu.VMEM((GATHERS, IDX_PER_GATHER), jnp.int32),
            pltpu.VMEM((IDX_PER_GATHER, EMB), jnp.float32),
            pltpu.VMEM((IDX_PER_GATHER, EMB), jnp.float32),
            pltpu.VMEM((ROWS_PER_WORKER, EMB), jnp.float32),
            pltpu.SemaphoreType.DMA,
            pltpu.SemaphoreType.DMA,
        ],
        compiler_params=pltpu.CompilerParams(use_tc_tiling_on_sc=False),
    )
    return fn(x, w)


def kernel(X, W):
    xi = X.astype(jnp.int32)
    j, m = xi // _TB, xi % _TB
    c, p = m // _QT, m % _QT
    x = (4 * (j * _QT + p) + c).reshape(NUM_WORKERS, GATHERS, IDX_PER_GATHER)
    w_lin = _prepare_table(W.T).reshape(4 * _OUT_ROWS, EMB)
    return _emb_mean(x, w_lin)
